# baseline (speedup 1.0000x reference)
import functools
import math

import jax
import jax.numpy as jnp
from jax import lax
from jax.experimental import pallas as pl
from jax.experimental.pallas import tpu as pltpu
from jax.experimental.pallas import tpu_sc as plsc

D_MODEL = 768
_SCALE = math.sqrt(D_MODEL)

_info = plsc.get_sparse_core_info()
_NC = _info.num_cores
_NS = _info.num_subcores
_L = _info.num_lanes
_NW = _NC * _NS

_CHUNK = 32
_HALF = _CHUNK // 2
_NG = 4


def _make_kernel(B: int):
    assert B % (_NW * _CHUNK) == 0
    b_per_w = B // _NW
    n_chunks = b_per_w // _CHUNK
    assert n_chunks % _NG == 0 and n_chunks >= 2 * _NG
    n_vecs = D_MODEL // _L

    mesh = plsc.VectorSubcoreMesh(core_axis_name="c", subcore_axis_name="s")

    @functools.partial(
        pl.kernel,
        mesh=mesh,
        out_type=jax.ShapeDtypeStruct((B, D_MODEL), jnp.float32),
        scratch_types=(
            [pltpu.VMEM((n_chunks, _CHUNK), jnp.int32)]
            + [pltpu.VMEM((_CHUNK, D_MODEL), jnp.float32)] * _NG
            + [pltpu.VMEM((_HALF, D_MODEL), jnp.float32)] * 2
            + [pltpu.SemaphoreType.DMA] * (_NG + 2)
        ),
    )
    def emb_kernel(table_hbm, x_hbm, out_hbm, idx_v, *rest):
        gbufs = rest[:_NG]
        obufs = rest[_NG:_NG + 2]
        gsems = rest[_NG + 2:2 * _NG + 2]
        osems = rest[2 * _NG + 2:2 * _NG + 4]

        wid = lax.axis_index("s") * _NC + lax.axis_index("c")
        base = wid * b_per_w

        pltpu.sync_copy(x_hbm.at[wid], idx_v)

        def issue_gather(c, g):
            pltpu.async_copy(table_hbm.at[idx_v.at[c]], gbufs[g], gsems[g])

        def wait_gather(g):
            pltpu.make_async_copy(
                table_hbm.at[idx_v.at[0]], gbufs[g], gsems[g]).wait()

        def issue_write(c, h):
            pltpu.async_copy(
                obufs[h],
                out_hbm.at[pl.ds(base + c * _CHUNK + h * _HALF, _HALF)],
                osems[h])

        def wait_write(h):
            pltpu.make_async_copy(
                obufs[h], out_hbm.at[pl.ds(base, _HALF)], osems[h]).wait()

        def scale_half(g, h):
            src = gbufs[g]
            dst = obufs[h]
            r0 = h * _HALF
            def row_body(r, carry):
                for j in range(n_vecs):
                    dst[(r, pl.ds(j * _L, _L))] = (
                        src[(r + r0, pl.ds(j * _L, _L))] * _SCALE)
                return carry
            lax.fori_loop(0, _HALF, row_body, 0)

        def process(c, g, first, issue_next):
            wait_gather(g)
            if not first:
                wait_write(0)
            scale_half(g, 0)
            issue_write(c, 0)
            if not first:
                wait_write(1)
            scale_half(g, 1)
            if issue_next:
                issue_gather(c + _NG, g)
            issue_write(c, 1)

        for g in range(_NG):
            issue_gather(g, g)
        process(0, 0, True, True)
        for c in range(1, _NG):
            process(c, c % _NG, False, True)

        def loop_body(i, carry):
            cc = _NG + _NG * i
            for g in range(_NG):
                process(cc + g, g, False, True)
            return carry
        lax.fori_loop(0, (n_chunks - 2 * _NG) // _NG, loop_body, 0)

        for c in range(n_chunks - _NG, n_chunks):
            process(c, c % _NG, False, False)
        wait_write(0)
        wait_write(1)

    return emb_kernel


def kernel(table, x):
    B = x.size
    x_blocked = x.reshape(_NW, B // _NW // _CHUNK, _CHUNK)
    out = _make_kernel(B)(table, x_blocked)
    return out.reshape(x.shape + (D_MODEL,))


# speedup vs baseline: 1.5085x; 1.5085x over previous
import functools
import math

import jax
import jax.numpy as jnp
from jax import lax
from jax.experimental import pallas as pl
from jax.experimental.pallas import tpu as pltpu
from jax.experimental.pallas import tpu_sc as plsc

D_MODEL = 768
_SCALE = math.sqrt(D_MODEL)

_info = plsc.get_sparse_core_info()
_NC = _info.num_cores
_NS = _info.num_subcores
_L = _info.num_lanes
_NW = _NC * _NS

_CHUNK = 32
_NG = 3
_NO = 2
_PERIOD = 6


def _make_kernel(B: int):
    assert B % (_NW * _CHUNK) == 0
    b_per_w = B // _NW
    n_chunks = b_per_w // _CHUNK
    n_vecs = D_MODEL // _L
    loop_iters = (n_chunks - 2 - _PERIOD) // _PERIOD
    loop_end = 2 + _PERIOD * loop_iters
    assert loop_end + _NG <= n_chunks

    mesh = plsc.VectorSubcoreMesh(core_axis_name="c", subcore_axis_name="s")

    @functools.partial(
        pl.kernel,
        mesh=mesh,
        out_type=jax.ShapeDtypeStruct((B, D_MODEL), jnp.float32),
        scratch_types=(
            [pltpu.VMEM((n_chunks, _CHUNK), jnp.int32)]
            + [pltpu.VMEM((_CHUNK, D_MODEL), jnp.float32)] * (_NG + _NO)
            + [pltpu.SemaphoreType.DMA] * (_NG + _NO)
        ),
    )
    def emb_kernel(table_hbm, x_hbm, out_hbm, idx_v, *rest):
        gbufs = rest[:_NG]
        obufs = rest[_NG:_NG + _NO]
        gsems = rest[_NG + _NO:2 * _NG + _NO]
        osems = rest[2 * _NG + _NO:2 * (_NG + _NO)]

        wid = lax.axis_index("s") * _NC + lax.axis_index("c")
        base = wid * b_per_w

        pltpu.sync_copy(x_hbm.at[wid], idx_v)

        def issue_gather(c, g):
            pltpu.async_copy(table_hbm.at[idx_v.at[c]], gbufs[g], gsems[g])

        def wait_gather(g):
            pltpu.make_async_copy(
                table_hbm.at[idx_v.at[0]], gbufs[g], gsems[g]).wait()

        def issue_write(c, o):
            pltpu.async_copy(
                obufs[o], out_hbm.at[pl.ds(base + c * _CHUNK, _CHUNK)],
                osems[o])

        def wait_write(o):
            pltpu.make_async_copy(
                obufs[o], out_hbm.at[pl.ds(base, _CHUNK)], osems[o]).wait()

        def scale(g, o):
            src = gbufs[g]
            dst = obufs[o]
            def row_body(r, carry):
                for j in range(n_vecs):
                    sl = (r, pl.ds(j * _L, _L))
                    dst[sl] = src[sl] * _SCALE
                return carry
            lax.fori_loop(0, _CHUNK, row_body, 0)

        def process(c, g, o, wait_w, issue_next):
            wait_gather(g)
            if wait_w:
                wait_write(o)
            scale(g, o)
            if issue_next:
                issue_gather(c + _NG, g)
            issue_write(c, o)

        for g in range(_NG):
            issue_gather(g, g)
        process(0, 0, 0, False, True)
        process(1, 1, 1, False, True)

        def loop_body(i, carry):
            cc = 2 + _PERIOD * i
            for j in range(_PERIOD):
                c = cc + j
                process(c, (2 + j) % _NG, j % _NO, True, True)
            return carry
        lax.fori_loop(0, loop_iters, loop_body, 0)

        for c in range(loop_end, n_chunks):
            process(c, c % _NG, c % _NO, True, c + _NG < n_chunks)
        for o in range(_NO):
            wait_write(o)

    return emb_kernel


def kernel(table, x):
    B = x.size
    x_blocked = x.reshape(_NW, B // _NW // _CHUNK, _CHUNK)
    out = _make_kernel(B)(table, x_blocked)
    return out.reshape(x.shape + (D_MODEL,))


# speedup vs baseline: 1.5317x; 1.0154x over previous
import functools
import math

import jax
import jax.numpy as jnp
from jax import lax
from jax.experimental import pallas as pl
from jax.experimental.pallas import tpu as pltpu
from jax.experimental.pallas import tpu_sc as plsc

D_MODEL = 768
_SCALE = math.sqrt(D_MODEL)

_info = plsc.get_sparse_core_info()
_NC = _info.num_cores
_NS = _info.num_subcores
_L = _info.num_lanes
_NW = _NC * _NS

_CHUNK = 32


def _make_kernel(B: int):
    assert B % (_NW * _CHUNK) == 0
    b_per_w = B // _NW
    n_chunks = b_per_w // _CHUNK
    assert n_chunks >= 4 and n_chunks % 2 == 0
    n_vecs = D_MODEL // _L

    mesh = plsc.VectorSubcoreMesh(core_axis_name="c", subcore_axis_name="s")

    @functools.partial(
        pl.kernel,
        mesh=mesh,
        out_type=jax.ShapeDtypeStruct((B, D_MODEL), jnp.float32),
        scratch_types=[
            pltpu.VMEM((n_chunks, _CHUNK), jnp.int32),
            pltpu.VMEM((_CHUNK, D_MODEL), jnp.float32),
            pltpu.VMEM((_CHUNK, D_MODEL), jnp.float32),
            pltpu.VMEM((_CHUNK, D_MODEL), jnp.float32),
            pltpu.VMEM((_CHUNK, D_MODEL), jnp.float32),
            pltpu.SemaphoreType.DMA,
            pltpu.SemaphoreType.DMA,
            pltpu.SemaphoreType.DMA,
            pltpu.SemaphoreType.DMA,
        ],
    )
    def emb_kernel(table_hbm, x_hbm, out_hbm, idx_v, gbuf0, gbuf1,
                   obuf0, obuf1, gsem0, gsem1, osem0, osem1):
        wid = lax.axis_index("s") * _NC + lax.axis_index("c")
        base = wid * b_per_w

        gbufs = (gbuf0, gbuf1)
        obufs = (obuf0, obuf1)
        gsems = (gsem0, gsem1)
        osems = (osem0, osem1)

        pltpu.sync_copy(x_hbm.at[wid], idx_v)

        def issue_gather(c, b):
            return pltpu.async_copy(
                table_hbm.at[idx_v.at[c]], gbufs[b], gsems[b])

        def wait_gather(b):
            pltpu.make_async_copy(
                table_hbm.at[idx_v.at[0]], gbufs[b], gsems[b]).wait()

        def issue_write(c, b):
            return pltpu.async_copy(
                obufs[b], out_hbm.at[pl.ds(base + c * _CHUNK, _CHUNK)],
                osems[b])

        def wait_write(b):
            pltpu.make_async_copy(
                obufs[b], out_hbm.at[pl.ds(base, _CHUNK)], osems[b]).wait()

        def scale(b):
            src = gbufs[b]
            dst = obufs[b]
            def row_body(r2, carry):
                for rr in range(2):
                    for j in range(n_vecs):
                        sl = (r2 * 2 + rr, pl.ds(j * _L, _L))
                        dst[sl] = src[sl] * _SCALE
                return carry
            lax.fori_loop(0, _CHUNK // 2, row_body, 0)

        issue_gather(0, 0)
        issue_gather(1, 1)
        for b in (0, 1):
            wait_gather(b)
            scale(b)
            issue_gather(b + 2, b)
            issue_write(b, b)

        def loop_body(i, carry):
            g = 2 + 2 * i
            for b in (0, 1):
                c = g + b
                wait_gather(b)
                wait_write(b)
                scale(b)
                issue_gather(c + 2, b)
                issue_write(c, b)
            return carry
        lax.fori_loop(0, (n_chunks - 4) // 2, loop_body, 0)

        for b in (0, 1):
            c = n_chunks - 2 + b
            wait_gather(b)
            wait_write(b)
            scale(b)
            issue_write(c, b)
        wait_write(0)
        wait_write(1)

    return emb_kernel


def kernel(table, x):
    B = x.size
    x_blocked = x.reshape(_NW, B // _NW // _CHUNK, _CHUNK)
    out = _make_kernel(B)(table, x_blocked)
    return out.reshape(x.shape + (D_MODEL,))


# speedup vs baseline: 1.5431x; 1.0075x over previous
import functools
import math

import jax
import jax.numpy as jnp
from jax import lax
from jax.experimental import pallas as pl
from jax.experimental.pallas import tpu as pltpu
from jax.experimental.pallas import tpu_sc as plsc

D_MODEL = 768
_SCALE = math.sqrt(D_MODEL)

_info = plsc.get_sparse_core_info()
_NC = _info.num_cores
_NS = _info.num_subcores
_L = _info.num_lanes
_NW = _NC * _NS

_CHUNK = 32


def _make_kernel(B: int):
    assert B % (_NW * _CHUNK) == 0
    b_per_w = B // _NW
    n_chunks = b_per_w // _CHUNK
    assert n_chunks >= 4 and n_chunks % 2 == 0
    n_vecs = D_MODEL // _L

    mesh = plsc.VectorSubcoreMesh(core_axis_name="c", subcore_axis_name="s")

    @functools.partial(
        pl.kernel,
        mesh=mesh,
        out_type=jax.ShapeDtypeStruct((B, D_MODEL), jnp.float32),
        scratch_types=[
            pltpu.VMEM((n_chunks, _CHUNK), jnp.int32),
            pltpu.VMEM((_CHUNK, D_MODEL), jnp.float32),
            pltpu.VMEM((_CHUNK, D_MODEL), jnp.float32),
            pltpu.VMEM((_CHUNK, D_MODEL), jnp.float32),
            pltpu.VMEM((_CHUNK, D_MODEL), jnp.float32),
            pltpu.SemaphoreType.DMA,
            pltpu.SemaphoreType.DMA,
            pltpu.SemaphoreType.DMA,
            pltpu.SemaphoreType.DMA,
        ],
    )
    def emb_kernel(table_hbm, x_hbm, out_hbm, idx_v, gbuf0, gbuf1,
                   obuf0, obuf1, gsem0, gsem1, osem0, osem1):
        wid = lax.axis_index("s") * _NC + lax.axis_index("c")
        base = wid * b_per_w

        gbufs = (gbuf0, gbuf1)
        obufs = (obuf0, obuf1)
        gsems = (gsem0, gsem1)
        osems = (osem0, osem1)

        pltpu.sync_copy(x_hbm.at[wid], idx_v)

        def issue_gather(c, b):
            return pltpu.async_copy(
                table_hbm.at[idx_v.at[c]], gbufs[b], gsems[b])

        def wait_gather(b):
            pltpu.make_async_copy(
                table_hbm.at[idx_v.at[0]], gbufs[b], gsems[b]).wait()

        def issue_write(c, b):
            return pltpu.async_copy(
                obufs[b], out_hbm.at[pl.ds(base + c * _CHUNK, _CHUNK)],
                osems[b])

        def wait_write(b):
            pltpu.make_async_copy(
                obufs[b], out_hbm.at[pl.ds(base, _CHUNK)], osems[b]).wait()

        def scale_rows(b, r0, nrows):
            src = gbufs[b]
            dst = obufs[b]
            def row_body(r, carry):
                for j in range(n_vecs):
                    sl = (r, pl.ds(j * _L, _L))
                    dst[sl] = src[sl] * _SCALE
                return carry
            lax.fori_loop(r0, r0 + nrows, row_body, 0)

        _H = _CHUNK // 2

        def issue_write_half(c, b, h):
            pltpu.async_copy(
                obufs[b].at[pl.ds(h * _H, _H)],
                out_hbm.at[pl.ds(base + c * _CHUNK + h * _H, _H)],
                osems[b])

        def scale_and_write(c, b):
            scale_rows(b, 0, _H)
            issue_write_half(c, b, 0)
            scale_rows(b, _H, _H)
            issue_write_half(c, b, 1)

        issue_gather(0, 0)
        issue_gather(1, 1)
        for b in (0, 1):
            wait_gather(b)
            scale_and_write(b, b)
            issue_gather(b + 2, b)

        def loop_body(i, carry):
            g = 2 + 2 * i
            for b in (0, 1):
                c = g + b
                wait_gather(b)
                wait_write(b)
                scale_and_write(c, b)
                issue_gather(c + 2, b)
            return carry
        lax.fori_loop(0, (n_chunks - 4) // 2, loop_body, 0)

        for b in (0, 1):
            c = n_chunks - 2 + b
            wait_gather(b)
            wait_write(b)
            scale_and_write(c, b)
        wait_write(0)
        wait_write(1)

    return emb_kernel


def kernel(table, x):
    B = x.size
    x_blocked = x.reshape(_NW, B // _NW // _CHUNK, _CHUNK)
    out = _make_kernel(B)(table, x_blocked)
    return out.reshape(x.shape + (D_MODEL,))


# speedup vs baseline: 1.5680x; 1.0161x over previous
import functools
import math

import jax
import jax.numpy as jnp
from jax import lax
from jax.experimental import pallas as pl
from jax.experimental.pallas import tpu as pltpu
from jax.experimental.pallas import tpu_sc as plsc

D_MODEL = 768
_SCALE = math.sqrt(D_MODEL)

_info = plsc.get_sparse_core_info()
_NC = _info.num_cores
_NS = _info.num_subcores
_L = _info.num_lanes
_NW = _NC * _NS

_CHUNK = 32


def _make_kernel(B: int):
    assert B % (_NW * _CHUNK) == 0
    b_per_w = B // _NW
    n_chunks = b_per_w // _CHUNK
    assert n_chunks >= 4 and n_chunks % 2 == 0
    n_vecs = D_MODEL // _L

    mesh = plsc.VectorSubcoreMesh(core_axis_name="c", subcore_axis_name="s")

    @functools.partial(
        pl.kernel,
        mesh=mesh,
        out_type=jax.ShapeDtypeStruct((B, D_MODEL), jnp.float32),
        scratch_types=[
            pltpu.VMEM((n_chunks, _CHUNK), jnp.int32),
            pltpu.VMEM((_CHUNK, D_MODEL), jnp.float32),
            pltpu.VMEM((_CHUNK, D_MODEL), jnp.float32),
            pltpu.VMEM((_CHUNK, D_MODEL), jnp.float32),
            pltpu.VMEM((_CHUNK, D_MODEL), jnp.float32),
            pltpu.SemaphoreType.DMA,
            pltpu.SemaphoreType.DMA,
            pltpu.SemaphoreType.DMA,
            pltpu.SemaphoreType.DMA,
        ],
    )
    def emb_kernel(table_hbm, x_hbm, out_hbm, idx_v, gbuf0, gbuf1,
                   obuf0, obuf1, gsem0, gsem1, osem0, osem1):
        wid = lax.axis_index("s") * _NC + lax.axis_index("c")
        base = wid * b_per_w

        gbufs = (gbuf0, gbuf1)
        obufs = (obuf0, obuf1)
        gsems = (gsem0, gsem1)
        osems = (osem0, osem1)

        pltpu.sync_copy(x_hbm.at[wid], idx_v)

        def issue_gather(c, b):
            return pltpu.async_copy(
                table_hbm.at[idx_v.at[c]], gbufs[b], gsems[b])

        def wait_gather(b):
            pltpu.make_async_copy(
                table_hbm.at[idx_v.at[0]], gbufs[b], gsems[b]).wait()

        def issue_write(c, b):
            return pltpu.async_copy(
                obufs[b], out_hbm.at[pl.ds(base + c * _CHUNK, _CHUNK)],
                osems[b])

        def wait_write(b):
            pltpu.make_async_copy(
                obufs[b], out_hbm.at[pl.ds(base, _CHUNK)], osems[b]).wait()

        def scale(b):
            src = gbufs[b]
            dst = obufs[b]
            def row_body(r, carry):
                for j in range(n_vecs):
                    sl = (r, pl.ds(j * _L, _L))
                    dst[sl] = src[sl] * _SCALE
                return carry
            lax.fori_loop(0, _CHUNK, row_body, 0)

        issue_gather(0, 0)
        issue_gather(1, 1)
        for b in (0, 1):
            wait_gather(b)
            scale(b)
            issue_gather(b + 2, b)
            issue_write(b, b)

        def loop_body(i, carry):
            g = 2 + 2 * i
            for b in (0, 1):
                c = g + b
                wait_gather(b)
                wait_write(b)
                scale(b)
                issue_gather(c + 2, b)
                issue_write(c, b)
            return carry
        lax.fori_loop(0, (n_chunks - 4) // 2, loop_body, 0)

        for b in (0, 1):
            c = n_chunks - 2 + b
            wait_gather(b)
            wait_write(b)
            scale(b)
            issue_write(c, b)
        wait_write(0)
        wait_write(1)

    return emb_kernel


def kernel(table, x):
    B = x.size
    x_blocked = x.reshape(_NW, B // _NW // _CHUNK, _CHUNK)
    out = _make_kernel(B)(table, x_blocked)
    return out.reshape(x.shape + (D_MODEL,))
